# baseline (speedup 1.0000x reference)
import jax
import jax.numpy as jnp
from jax.experimental import pallas as pl
from jax.experimental.pallas import tpu as pltpu

_DIM = 64
_Q = 16
_EPS = 1e-8


def _scan_body(q_ref, m_ref, w_ref, b_ref, out_ref, bsim_ref, bmem_ref):
    i = pl.program_id(0)
    nblk = pl.num_programs(0)
    blk = m_ref.shape[0]

    @pl.when(i == 0)
    def _init():
        bsim_ref[...] = jnp.full_like(bsim_ref, -jnp.inf)
        bmem_ref[...] = jnp.zeros_like(bmem_ref)

    q = q_ref[...]
    qn = q / (jnp.sqrt(jnp.sum(q * q, axis=1, keepdims=True)) + _EPS)

    m = m_ref[...]
    s = jax.lax.dot_general(qn, m, (((1,), (1,)), ((), ())),
                            preferred_element_type=jnp.float32)
    rn = 1.0 / (jnp.sqrt(jnp.sum(m * m, axis=1)) + _EPS)
    sims = s * rn[None, :]

    bmax = jnp.max(sims, axis=1, keepdims=True)
    col = jax.lax.broadcasted_iota(jnp.int32, sims.shape, 1)
    lidx = jnp.min(jnp.where(sims >= bmax, col, blk), axis=1, keepdims=True)
    onehot = (col == lidx).astype(jnp.float32)
    rows = jax.lax.dot_general(onehot, m, (((1,), (0,)), ((), ())),
                               preferred_element_type=jnp.float32)

    upd = bmax > bsim_ref[...]
    bsim_ref[...] = jnp.where(upd, bmax, bsim_ref[...])
    bmem_ref[...] = jnp.where(upd, rows, bmem_ref[...])

    @pl.when(i == nblk - 1)
    def _final():
        bm = bmem_ref[...]
        o = jax.lax.dot_general(bm, w_ref[...], (((1,), (1,)), ((), ())),
                                preferred_element_type=jnp.float32)
        o = o + b_ref[...]
        gate = (bsim_ref[...] > 0.6).astype(jnp.float32)
        out_ref[...] = o * gate


def kernel(query, memories, W_dec, b_dec):
    cap = memories.shape[0]
    blk = 20000
    grid = cap // blk
    b2 = b_dec.reshape(1, _DIM)

    out = pl.pallas_call(
        _scan_body,
        grid=(grid,),
        in_specs=[
            pl.BlockSpec((_Q, _DIM), lambda i: (0, 0)),
            pl.BlockSpec((blk, _DIM), lambda i: (i, 0)),
            pl.BlockSpec((_DIM, _DIM), lambda i: (0, 0)),
            pl.BlockSpec((1, _DIM), lambda i: (0, 0)),
        ],
        out_specs=pl.BlockSpec((_Q, _DIM), lambda i: (0, 0)),
        out_shape=jax.ShapeDtypeStruct((_Q, _DIM), jnp.float32),
        scratch_shapes=[
            pltpu.VMEM((_Q, 1), jnp.float32),
            pltpu.VMEM((_Q, _DIM), jnp.float32),
        ],
        compiler_params=pltpu.CompilerParams(
            dimension_semantics=("arbitrary",),
        ),
    )(query, memories, W_dec, b2)
    return out
